# baseline (speedup 1.0000x reference)
import jax
import jax.numpy as jnp
from jax import lax
from jax.experimental import pallas as pl
from jax.experimental.pallas import tpu as pltpu
from jax.experimental.pallas import tpu_sc as plsc

_B = 1024
_V = 100000
_DS = 300
_DSP = 384
_DC = 1024
_WS = 0.1
_WT = 0.1
_K = 10
_TV = 1024
_NSTEP = 98
_VP = _NSTEP * _TV
_CW = 128
_NCH = _VP // _CW
_CPS = _TV // _CW


def _l2n(x):
    return x / jnp.sqrt(jnp.sum(x * x, axis=1, keepdims=True))


def _transpose_body(stT_ref, out_ref):
    out_ref[:, :_DS] = lax.transpose(stT_ref[...], (1, 0))


def _prep_body(est_ref, ecs_ref, ect_ref, w1_ref, b1_ref, w2_ref, b2_ref,
               w3_ref, b3_ref, w4_ref, b4_ref, tgt_ref, cs_ref):
    stn = _l2n(est_ref[:, :_DS])
    cs = jnp.tanh(lax.dot_general(ecs_ref[...], w1_ref[...],
                                  (((1,), (1,)), ((), ()))) + b1_ref[...])
    cs = jnp.tanh(lax.dot_general(cs, w3_ref[...],
                                  (((1,), (1,)), ((), ()))) + b3_ref[...])
    cs = _l2n(cs)
    ct = jnp.tanh(lax.dot_general(ect_ref[...], w2_ref[...],
                                  (((1,), (1,)), ((), ()))) + b2_ref[...])
    ct = jnp.tanh(lax.dot_general(ct, w4_ref[...],
                                  (((1,), (1,)), ((), ()))) + b4_ref[...])
    ct = _l2n(ct)
    tgt_ref[...] = stn + _WT * ct
    cs_ref[...] = cs


def _scores_body(tgt_ref, ctx_ref, staT_ref, w1_ref, b1_ref, w3_ref, b3_ref,
                 s_ref, m_ref, srm_ref):
    i = pl.program_id(0)
    h = jnp.tanh(lax.dot_general(ctx_ref[...], w1_ref[...],
                                 (((1,), (1,)), ((), ()))) + b1_ref[...])
    h = jnp.tanh(lax.dot_general(h, w3_ref[...],
                                 (((1,), (1,)), ((), ()))) + b3_ref[...])
    h = _l2n(h)
    tgt = tgt_ref[...]
    staT = staT_ref[...]
    keyT = staT + _WS * lax.transpose(h, (1, 0))
    s = lax.dot_general(tgt, keyT, (((1,), (0,)), ((), ())))

    @pl.when(i < _NSTEP - 1)
    def _full():
        s3 = jnp.transpose(s.reshape(_B, _CPS, _CW), (1, 0, 2))
        s_ref[...] = s3.reshape(_CPS * _B, _CW)
        m_ref[...] = jnp.max(s3, axis=2)

    @pl.when(i == _NSTEP - 1)
    def _tail():
        col = i * _TV + lax.broadcasted_iota(jnp.int32, (_B, _TV), 1)
        sm = jnp.where(col < _V, s, -jnp.inf)
        s3 = jnp.transpose(sm.reshape(_B, _CPS, _CW), (1, 0, 2))
        s_ref[...] = s3.reshape(_CPS * _B, _CW)
        m_ref[...] = jnp.max(s3, axis=2)
    srm_ref[:, :_DS] = lax.transpose(staT, (1, 0))


def _select_body(m_ref, idx_ref):
    w = m_ref[...]
    ri = lax.broadcasted_iota(jnp.int32, (_NCH, _B), 0)
    ki = lax.broadcasted_iota(jnp.int32, (16, _B), 0)
    acc = jnp.zeros((16, _B), jnp.int32)
    for k in range(_K):
        mx = jnp.max(w, axis=0, keepdims=True)
        pos = jnp.min(jnp.where(w == mx, ri, _NCH), axis=0, keepdims=True)
        acc = jnp.where(ki == k, pos, acc)
        w = jnp.where(ri == pos, -jnp.inf, w)
    colbase = lax.broadcasted_iota(jnp.int32, (16, _B), 1)
    idx_ref[...] = acc * _B + colbase


def _final_body(g_ref, ess_ref, tgt_ref, cs_ref, out_ref):
    ssn = _l2n(ess_ref[:, :_DS])
    tgt = tgt_ref[...]
    src = ssn + _WS * cs_ref[...]
    dot2 = 2.0 * jnp.sum(src * tgt, axis=1, keepdims=True)
    w = g_ref[...]
    ci = (lax.broadcasted_iota(jnp.int32, (_K, _B, _CW), 0) * _CW
          + lax.broadcasted_iota(jnp.int32, (_K, _B, _CW), 2))
    big = _K * _CW
    ssum = jnp.zeros((_B, 1), jnp.float32)
    for k in range(_K):
        mx = jnp.max(jnp.max(w, axis=0), axis=1, keepdims=True)
        ssum = ssum + mx
        wh = jnp.where(w == mx[None], ci, big)
        pos = jnp.min(jnp.min(wh, axis=0), axis=1, keepdims=True)
        w = jnp.where(ci == pos[None], -jnp.inf, w)
    out_ref[...] = dot2 - ssum * (1.0 / _K)


_NW = 32
_GB = _K * _B
_BPW = _GB // _NW
_EBPW = _B // _NW


def _early_gather_body(t1, t2, t3, i1, i2, i3, o1, o2, o3,
                       iv1, iv2, iv3, r1, r2, r3, sem):
    wid = lax.axis_index("s") * 2 + lax.axis_index("c")
    base = wid * _EBPW
    pltpu.sync_copy(i1.at[pl.ds(base, _EBPW)], iv1)
    pltpu.sync_copy(i2.at[pl.ds(base, _EBPW)], iv2)
    pltpu.sync_copy(i3.at[pl.ds(base, _EBPW)], iv3)
    c1 = pltpu.async_copy(t1.at[iv1], r1, sem)
    c2 = pltpu.async_copy(t2.at[iv2], r2, sem)
    c3 = pltpu.async_copy(t3.at[iv3], r3, sem)
    c1.wait()
    c2.wait()
    c3.wait()
    pltpu.sync_copy(r1, o1.at[pl.ds(base, _EBPW)])
    pltpu.sync_copy(r2, o2.at[pl.ds(base, _EBPW)])
    pltpu.sync_copy(r3, o3.at[pl.ds(base, _EBPW)])


def _early_gather(csw, ctw, st_rm, cid, ctid, tid):
    f32 = jnp.float32
    return pl.kernel(
        _early_gather_body,
        out_type=(jax.ShapeDtypeStruct((_B, _DC), f32),
                  jax.ShapeDtypeStruct((_B, _DC), f32),
                  jax.ShapeDtypeStruct((_B, _DSP), f32)),
        mesh=plsc.VectorSubcoreMesh(core_axis_name="c", subcore_axis_name="s"),
        scratch_types=[
            pltpu.VMEM((_EBPW,), jnp.int32),
            pltpu.VMEM((_EBPW,), jnp.int32),
            pltpu.VMEM((_EBPW,), jnp.int32),
            pltpu.VMEM((_EBPW, _DC), f32),
            pltpu.VMEM((_EBPW, _DC), f32),
            pltpu.VMEM((_EBPW, _DSP), f32),
            pltpu.SemaphoreType.DMA,
        ],
    )(csw, ctw, st_rm, cid, ctid, tid)


def _late_gather_body(tbl, srm, fidx, sidx, oc, oe, iv, ivs, rc, re, sem):
    wid = lax.axis_index("s") * 2 + lax.axis_index("c")
    base = wid * _BPW
    base_e = wid * _EBPW
    pltpu.sync_copy(fidx.at[pl.ds(base, _BPW)], iv)
    pltpu.sync_copy(sidx.at[pl.ds(base_e, _EBPW)], ivs)
    c1 = pltpu.async_copy(tbl.at[iv], rc, sem)
    c2 = pltpu.async_copy(srm.at[ivs], re, sem)
    c1.wait()
    c2.wait()
    pltpu.sync_copy(rc, oc.at[pl.ds(base, _BPW)])
    pltpu.sync_copy(re, oe.at[pl.ds(base_e, _EBPW)])


def _late_gather(tbl, srm, fidx, sidx):
    f32 = jnp.float32
    return pl.kernel(
        _late_gather_body,
        out_type=(jax.ShapeDtypeStruct((_GB, _CW), f32),
                  jax.ShapeDtypeStruct((_B, _DSP), f32)),
        mesh=plsc.VectorSubcoreMesh(core_axis_name="c", subcore_axis_name="s"),
        scratch_types=[
            pltpu.VMEM((_BPW,), jnp.int32),
            pltpu.VMEM((_EBPW,), jnp.int32),
            pltpu.VMEM((_BPW, _CW), f32),
            pltpu.VMEM((_EBPW, _DSP), f32),
            pltpu.SemaphoreType.DMA,
        ],
    )(tbl, srm, fidx, sidx)


def kernel(static_src_id, context_src_id, static_tgt_id, context_tgt_id,
           static_src_W, static_tgt_W, context_src_W, context_tgt_W,
           W1, b1, W2, b2, W3, b3, W4, b4):
    f32 = jnp.float32
    sid = jnp.asarray(static_src_id, jnp.int32)
    cid = jnp.asarray(context_src_id, jnp.int32)
    tid = jnp.asarray(static_tgt_id, jnp.int32)
    ctid = jnp.asarray(context_tgt_id, jnp.int32)

    ssT = static_src_W.T
    stT = static_tgt_W.T

    st_rm = pl.pallas_call(
        _transpose_body,
        grid=(_NSTEP,),
        in_specs=[pl.BlockSpec((_DS, _TV), lambda i: (0, i))],
        out_specs=pl.BlockSpec((_TV, _DSP), lambda i: (i, 0)),
        out_shape=jax.ShapeDtypeStruct((_VP, _DSP), f32),
        compiler_params=pltpu.CompilerParams(
            dimension_semantics=("arbitrary",)),
    )(stT)

    ecs, ect, est = _early_gather(context_src_W, context_tgt_W, st_rm,
                                  cid, ctid, tid)

    b1r = b1.reshape(1, _DS)
    b2r = b2.reshape(1, _DS)
    b3r = b3.reshape(1, _DS)
    b4r = b4.reshape(1, _DS)

    tgt, csn = pl.pallas_call(
        _prep_body,
        out_shape=(jax.ShapeDtypeStruct((_B, _DS), f32),
                   jax.ShapeDtypeStruct((_B, _DS), f32)),
    )(est, ecs, ect, W1, b1r, W2, b2r, W3, b3r, W4, b4r)

    scores, mchunk, ss_rm = pl.pallas_call(
        _scores_body,
        grid=(_NSTEP,),
        in_specs=[
            pl.BlockSpec((_B, _DS), lambda i: (0, 0)),
            pl.BlockSpec((_TV, _DC), lambda i: (i, 0)),
            pl.BlockSpec((_DS, _TV), lambda i: (0, i)),
            pl.BlockSpec((_DS, _DC), lambda i: (0, 0)),
            pl.BlockSpec((1, _DS), lambda i: (0, 0)),
            pl.BlockSpec((_DS, _DS), lambda i: (0, 0)),
            pl.BlockSpec((1, _DS), lambda i: (0, 0)),
        ],
        out_specs=[
            pl.BlockSpec((_CPS * _B, _CW), lambda i: (i, 0)),
            pl.BlockSpec((_CPS, _B), lambda i: (i, 0)),
            pl.BlockSpec((_TV, _DSP), lambda i: (i, 0)),
        ],
        out_shape=[jax.ShapeDtypeStruct((_NCH * _B, _CW), f32),
                   jax.ShapeDtypeStruct((_NCH, _B), f32),
                   jax.ShapeDtypeStruct((_VP, _DSP), f32)],
        compiler_params=pltpu.CompilerParams(
            dimension_semantics=("arbitrary",)),
    )(tgt, context_src_W, ssT, W1, b1r, W3, b3r)

    idx = pl.pallas_call(
        _select_body,
        out_shape=jax.ShapeDtypeStruct((16, _B), jnp.int32),
    )(mchunk)

    flat = idx[:_K].reshape(_K * _B)
    cand, ess = _late_gather(scores, ss_rm, flat, sid)

    out = pl.pallas_call(
        _final_body,
        out_shape=jax.ShapeDtypeStruct((_B, 1), f32),
    )(cand.reshape(_K, _B, _CW), ess, tgt, csn)
    return out.reshape(_B)


# speedup vs baseline: 1.2757x; 1.2757x over previous
import jax
import jax.numpy as jnp
from jax import lax
from jax.experimental import pallas as pl
from jax.experimental.pallas import tpu as pltpu
from jax.experimental.pallas import tpu_sc as plsc

_B = 1024
_V = 100000
_DS = 300
_DSP = 384
_DC = 1024
_WS = 0.1
_WT = 0.1
_K = 10
_TV = 2048
_NSTEP = 49
_VP = _NSTEP * _TV
_CW = 128
_NCH = _VP // _CW
_CPS = _TV // _CW


def _l2n(x):
    return x / jnp.sqrt(jnp.sum(x * x, axis=1, keepdims=True))


def _transpose_body(stT_ref, out_ref):
    out_ref[:, :_DS] = lax.transpose(stT_ref[...], (1, 0))


def _prep_body(est_ref, ecs_ref, ect_ref, w1_ref, b1_ref, w2_ref, b2_ref,
               w3_ref, b3_ref, w4_ref, b4_ref, tgt_ref, cs_ref):
    stn = _l2n(est_ref[:, :_DS])
    cs = jnp.tanh(lax.dot_general(ecs_ref[...], w1_ref[...],
                                  (((1,), (1,)), ((), ()))) + b1_ref[...])
    cs = jnp.tanh(lax.dot_general(cs, w3_ref[...],
                                  (((1,), (1,)), ((), ()))) + b3_ref[...])
    cs = _l2n(cs)
    ct = jnp.tanh(lax.dot_general(ect_ref[...], w2_ref[...],
                                  (((1,), (1,)), ((), ()))) + b2_ref[...])
    ct = jnp.tanh(lax.dot_general(ct, w4_ref[...],
                                  (((1,), (1,)), ((), ()))) + b4_ref[...])
    ct = _l2n(ct)
    tgt_ref[...] = stn + _WT * ct
    cs_ref[...] = cs


def _scores_body(tgt_ref, ctx_ref, staT_ref, w1_ref, b1_ref, w3_ref, b3_ref,
                 s_ref, m_ref, srm_ref):
    i = pl.program_id(0)
    h = jnp.tanh(lax.dot_general(ctx_ref[...], w1_ref[...],
                                 (((1,), (1,)), ((), ()))) + b1_ref[...])
    h = jnp.tanh(lax.dot_general(h, w3_ref[...],
                                 (((1,), (1,)), ((), ()))) + b3_ref[...])
    h = _l2n(h)
    tgt = tgt_ref[...]
    staT = staT_ref[...]
    keyT = staT + _WS * lax.transpose(h, (1, 0))
    s = lax.dot_general(tgt, keyT, (((1,), (0,)), ((), ())))
    col = i * _TV + lax.broadcasted_iota(jnp.int32, (_B, _TV), 1)
    sm = jnp.where(col < _V, s, -jnp.inf)
    s3 = jnp.transpose(sm.reshape(_B, _CPS, _CW), (1, 0, 2))
    s_ref[...] = s3.reshape(_CPS * _B, _CW)
    m_ref[...] = jnp.max(s3, axis=2)
    srm_ref[:, :_DS] = lax.transpose(staT, (1, 0))


def _select_body(m_ref, idx_ref):
    w = m_ref[...]
    ri = lax.broadcasted_iota(jnp.int32, (_NCH, _B), 0)
    ki = lax.broadcasted_iota(jnp.int32, (16, _B), 0)
    acc = jnp.zeros((16, _B), jnp.int32)
    for k in range(_K):
        mx = jnp.max(w, axis=0, keepdims=True)
        pos = jnp.min(jnp.where(w == mx, ri, _NCH), axis=0, keepdims=True)
        acc = jnp.where(ki == k, pos, acc)
        w = jnp.where(ri == pos, -jnp.inf, w)
    colbase = lax.broadcasted_iota(jnp.int32, (16, _B), 1)
    idx_ref[...] = acc * _B + colbase


def _final_body(g_ref, ess_ref, tgt_ref, cs_ref, out_ref):
    ssn = _l2n(ess_ref[:, :_DS])
    tgt = tgt_ref[...]
    src = ssn + _WS * cs_ref[...]
    dot2 = 2.0 * jnp.sum(src * tgt, axis=1, keepdims=True)
    w = g_ref[...]
    ci = (lax.broadcasted_iota(jnp.int32, (_K, _B, _CW), 0) * _CW
          + lax.broadcasted_iota(jnp.int32, (_K, _B, _CW), 2))
    big = _K * _CW
    ssum = jnp.zeros((_B, 1), jnp.float32)
    for k in range(_K):
        mx = jnp.max(jnp.max(w, axis=0), axis=1, keepdims=True)
        ssum = ssum + mx
        wh = jnp.where(w == mx[None], ci, big)
        pos = jnp.min(jnp.min(wh, axis=0), axis=1, keepdims=True)
        w = jnp.where(ci == pos[None], -jnp.inf, w)
    out_ref[...] = dot2 - ssum * (1.0 / _K)


_NW = 32
_GB = _K * _B
_BPW = _GB // _NW
_EBPW = _B // _NW


def _early_gather_body(t1, t2, t3, i1, i2, i3, o1, o2, o3,
                       iv1, iv2, iv3, r1, r2, r3, sem):
    wid = lax.axis_index("s") * 2 + lax.axis_index("c")
    base = wid * _EBPW
    pltpu.sync_copy(i1.at[pl.ds(base, _EBPW)], iv1)
    pltpu.sync_copy(i2.at[pl.ds(base, _EBPW)], iv2)
    pltpu.sync_copy(i3.at[pl.ds(base, _EBPW)], iv3)
    c1 = pltpu.async_copy(t1.at[iv1], r1, sem)
    c2 = pltpu.async_copy(t2.at[iv2], r2, sem)
    c3 = pltpu.async_copy(t3.at[iv3], r3, sem)
    c1.wait()
    c2.wait()
    c3.wait()
    pltpu.sync_copy(r1, o1.at[pl.ds(base, _EBPW)])
    pltpu.sync_copy(r2, o2.at[pl.ds(base, _EBPW)])
    pltpu.sync_copy(r3, o3.at[pl.ds(base, _EBPW)])


def _early_gather(csw, ctw, st_rm, cid, ctid, tid):
    f32 = jnp.float32
    return pl.kernel(
        _early_gather_body,
        out_type=(jax.ShapeDtypeStruct((_B, _DC), f32),
                  jax.ShapeDtypeStruct((_B, _DC), f32),
                  jax.ShapeDtypeStruct((_B, _DSP), f32)),
        mesh=plsc.VectorSubcoreMesh(core_axis_name="c", subcore_axis_name="s"),
        scratch_types=[
            pltpu.VMEM((_EBPW,), jnp.int32),
            pltpu.VMEM((_EBPW,), jnp.int32),
            pltpu.VMEM((_EBPW,), jnp.int32),
            pltpu.VMEM((_EBPW, _DC), f32),
            pltpu.VMEM((_EBPW, _DC), f32),
            pltpu.VMEM((_EBPW, _DSP), f32),
            pltpu.SemaphoreType.DMA,
        ],
    )(csw, ctw, st_rm, cid, ctid, tid)


def _late_gather_body(tbl, srm, fidx, sidx, oc, oe, iv, ivs, rc, re, sem):
    wid = lax.axis_index("s") * 2 + lax.axis_index("c")
    base = wid * _BPW
    base_e = wid * _EBPW
    pltpu.sync_copy(fidx.at[pl.ds(base, _BPW)], iv)
    pltpu.sync_copy(sidx.at[pl.ds(base_e, _EBPW)], ivs)
    c1 = pltpu.async_copy(tbl.at[iv], rc, sem)
    c2 = pltpu.async_copy(srm.at[ivs], re, sem)
    c1.wait()
    c2.wait()
    pltpu.sync_copy(rc, oc.at[pl.ds(base, _BPW)])
    pltpu.sync_copy(re, oe.at[pl.ds(base_e, _EBPW)])


def _late_gather(tbl, srm, fidx, sidx):
    f32 = jnp.float32
    return pl.kernel(
        _late_gather_body,
        out_type=(jax.ShapeDtypeStruct((_GB, _CW), f32),
                  jax.ShapeDtypeStruct((_B, _DSP), f32)),
        mesh=plsc.VectorSubcoreMesh(core_axis_name="c", subcore_axis_name="s"),
        scratch_types=[
            pltpu.VMEM((_BPW,), jnp.int32),
            pltpu.VMEM((_EBPW,), jnp.int32),
            pltpu.VMEM((_BPW, _CW), f32),
            pltpu.VMEM((_EBPW, _DSP), f32),
            pltpu.SemaphoreType.DMA,
        ],
    )(tbl, srm, fidx, sidx)


def kernel(static_src_id, context_src_id, static_tgt_id, context_tgt_id,
           static_src_W, static_tgt_W, context_src_W, context_tgt_W,
           W1, b1, W2, b2, W3, b3, W4, b4):
    f32 = jnp.float32
    sid = jnp.asarray(static_src_id, jnp.int32)
    cid = jnp.asarray(context_src_id, jnp.int32)
    tid = jnp.asarray(static_tgt_id, jnp.int32)
    ctid = jnp.asarray(context_tgt_id, jnp.int32)

    ssT = static_src_W.T
    stT = static_tgt_W.T

    st_rm = pl.pallas_call(
        _transpose_body,
        grid=(_NSTEP,),
        in_specs=[pl.BlockSpec((_DS, _TV), lambda i: (0, i))],
        out_specs=pl.BlockSpec((_TV, _DSP), lambda i: (i, 0)),
        out_shape=jax.ShapeDtypeStruct((_VP, _DSP), f32),
        compiler_params=pltpu.CompilerParams(
            dimension_semantics=("arbitrary",)),
    )(stT)

    ecs, ect, est = _early_gather(context_src_W, context_tgt_W, st_rm,
                                  cid, ctid, tid)

    b1r = b1.reshape(1, _DS)
    b2r = b2.reshape(1, _DS)
    b3r = b3.reshape(1, _DS)
    b4r = b4.reshape(1, _DS)

    tgt, csn = pl.pallas_call(
        _prep_body,
        out_shape=(jax.ShapeDtypeStruct((_B, _DS), f32),
                   jax.ShapeDtypeStruct((_B, _DS), f32)),
    )(est, ecs, ect, W1, b1r, W2, b2r, W3, b3r, W4, b4r)

    scores, mchunk, ss_rm = pl.pallas_call(
        _scores_body,
        grid=(_NSTEP,),
        in_specs=[
            pl.BlockSpec((_B, _DS), lambda i: (0, 0)),
            pl.BlockSpec((_TV, _DC), lambda i: (i, 0)),
            pl.BlockSpec((_DS, _TV), lambda i: (0, i)),
            pl.BlockSpec((_DS, _DC), lambda i: (0, 0)),
            pl.BlockSpec((1, _DS), lambda i: (0, 0)),
            pl.BlockSpec((_DS, _DS), lambda i: (0, 0)),
            pl.BlockSpec((1, _DS), lambda i: (0, 0)),
        ],
        out_specs=[
            pl.BlockSpec((_CPS * _B, _CW), lambda i: (i, 0)),
            pl.BlockSpec((_CPS, _B), lambda i: (i, 0)),
            pl.BlockSpec((_TV, _DSP), lambda i: (i, 0)),
        ],
        out_shape=[jax.ShapeDtypeStruct((_NCH * _B, _CW), f32),
                   jax.ShapeDtypeStruct((_NCH, _B), f32),
                   jax.ShapeDtypeStruct((_VP, _DSP), f32)],
        compiler_params=pltpu.CompilerParams(
            dimension_semantics=("arbitrary",)),
    )(tgt, context_src_W, ssT, W1, b1r, W3, b3r)

    idx = pl.pallas_call(
        _select_body,
        out_shape=jax.ShapeDtypeStruct((16, _B), jnp.int32),
    )(mchunk)

    flat = idx[:_K].reshape(_K * _B)
    cand, ess = _late_gather(scores, ss_rm, flat, sid)

    out = pl.pallas_call(
        _final_body,
        out_shape=jax.ShapeDtypeStruct((_B, 1), f32),
    )(cand.reshape(_K, _B, _CW), ess, tgt, csn)
    return out.reshape(_B)
